# baseline (speedup 1.0000x reference)
import functools

import jax
import jax.numpy as jnp
from jax import lax
from jax.experimental import pallas as pl
from jax.experimental.pallas import tpu as pltpu
from jax.experimental.pallas import tpu_sc as plsc

_VOCAB = 32
_D = 16
_NT = 3
_NR = 8
_NW = 32
_LANES = 16
_FC = 2 * _NT * 4 * 7
_ZR = 14 * _D


def _make_sc_main(B):
    chunk = B // _NW
    ngrp = chunk // _LANES
    mesh = plsc.VectorSubcoreMesh(core_axis_name="c", subcore_axis_name="s")

    @functools.partial(
        pl.kernel,
        mesh=mesh,
        compiler_params=pltpu.CompilerParams(use_tc_tiling_on_sc=True,
                                             needs_layout_passes=False),
        out_type=jax.ShapeDtypeStruct((B, _ZR), jnp.float32),
        scratch_types=[
            pltpu.VMEM((chunk, 2), jnp.int32),
            pltpu.VMEM((chunk, _FC), jnp.int32),
            pltpu.VMEM((50, _D), jnp.float32),
            pltpu.VMEM((_VOCAB * _D,), jnp.float32),
            pltpu.VMEM((2 * _VOCAB,), jnp.float32),
            pltpu.VMEM((chunk, _ZR), jnp.float32),
        ],
    )
    def sc_main(nodes_hbm, feats_hbm, wpk_hbm, zc_hbm,
                nodes_v, feats_v, wpk_v, we_v, f12_v, zc_v):
        wid = lax.axis_index("s") * 2 + lax.axis_index("c")
        base = wid * chunk
        with jax.named_scope("stage_in"):
            pltpu.sync_copy(nodes_hbm.at[pl.ds(base, chunk)], nodes_v)
            pltpu.sync_copy(feats_hbm.at[pl.ds(base, chunk)], feats_v)
            pltpu.sync_copy(wpk_hbm, wpk_v)

        lane = lax.iota(jnp.int32, _LANES)
        zero16 = jnp.zeros((_LANES,), jnp.float32)

        def spl(x):
            return jnp.full((_LANES,), x, jnp.int32)

        tb_scope = jax.named_scope("table_build")
        tb_scope.__enter__()
        vvec = [lane, lane + _LANES]
        f1h = [zero16, zero16]
        f2h = [zero16, zero16]
        wgat_rows = [wpk_v[_VOCAB + k] for k in range(_D)]
        a1row = wpk_v[48]
        a2row = wpk_v[49]
        embcol = [[plsc.load_gather(wpk_v, [vvec[h], spl(k)]) for h in range(2)]
                  for k in range(_D)]
        for d in range(_D):
            a1d = a1row[d]
            a2d = a2row[d]
            for h in range(2):
                acc = embcol[0][h] * wgat_rows[0][d]
                for k in range(1, _D):
                    acc = acc + embcol[k][h] * wgat_rows[k][d]
                plsc.store_scatter(we_v, [vvec[h] * _D + d], acc)
                f1h[h] = f1h[h] + acc * a1d
                f2h[h] = f2h[h] + acc * a2d
        half = [lane, lane + _LANES]
        for h in range(2):
            plsc.store_scatter(f12_v, [half[h]], f1h[h])
            plsc.store_scatter(f12_v, [half[h] + _VOCAB], f2h[h])
        tb_scope.__exit__(None, None, None)

        def wtree8(w, xs):
            p = [w[r] * xs[r] for r in range(8)]
            return ((p[0] + p[1]) + (p[2] + p[3])) + \
                   ((p[4] + p[5]) + (p[6] + p[7]))

        def tree8(xs):
            return ((xs[0] + xs[1]) + (xs[2] + xs[3])) + \
                   ((xs[4] + xs[5]) + (xs[6] + xs[7]))

        with jax.named_scope("node_emb"):
         for g in range(ngrp):
            col = g * _LANES + lane
            for i in range(2):
                nid = plsc.load_gather(nodes_v, [col, spl(i)])
                for d in range(_D):
                    ne = plsc.load_gather(wpk_v, [nid, spl(d)])
                    plsc.store_scatter(zc_v, [col, spl(192 + i * _D + d)], ne)

        @plsc.parallel_loop(0, ngrp * 2 * _NT, unroll=2)
        def body(k, carry=None):
            g = k // (2 * _NT)
            it = k % (2 * _NT)
            i = it // _NT
            col = g * _LANES + lane
            fb = it * 28

            ids8 = [plsc.load_gather(nodes_v, [col, spl(i)])]
            for r in range(1, _NR):
                ids8.append(plsc.load_gather(feats_v, [col, spl(fb + r - 1)]))

            f1_0 = plsc.load_gather(f12_v, [ids8[0]])
            f2 = [plsc.load_gather(f12_v, [ids8[r] + _VOCAB])
                  for r in range(_NR)]
            e = [jnp.where(x >= 0.0, x, 0.2 * x) for x in
                 [f1_0 + f2r for f2r in f2]]
            m = jnp.maximum(jnp.maximum(jnp.maximum(e[0], e[1]),
                                        jnp.maximum(e[2], e[3])),
                            jnp.maximum(jnp.maximum(e[4], e[5]),
                                        jnp.maximum(e[6], e[7])))
            ex = [jnp.exp(er - m) for er in e]
            inv = 1.0 / tree8(ex)
            attn = [exr * inv for exr in ex]

            cids = [ids8[1]]
            for p in range(1, 4):
                cids.append(plsc.load_gather(feats_v, [col, spl(fb + p * 7)]))
            cb = [plsc.load_gather(wpk_v, [c, spl(0)]) != 0.0 for c in cids]
            cacc = jnp.logical_or(jnp.logical_or(cb[0], cb[1]),
                                  jnp.logical_or(cb[2], cb[3]))

            base_r = [ids8[r] * _D for r in range(_NR)]
            row0 = spl(it * _D)
            for d in range(_D):
                rows = [plsc.load_gather(we_v, [base_r[r] + d])
                        for r in range(_NR)]
                z0d = jnp.where(cacc, wtree8(attn, rows), zero16)
                zmd = jnp.where(cacc, tree8(rows) * 0.125, zero16)
                plsc.store_scatter(zc_v, [col, row0 + d], z0d)
                plsc.store_scatter(zc_v, [col, row0 + (96 + d)], zmd)

        with jax.named_scope("stage_out"):
            pltpu.sync_copy(zc_v, zc_hbm.at[pl.ds(base, chunk)])

    return sc_main


def _make_post(B):
    def post_body(zc_ref, wsem_ref, bsem_ref, qsem_ref, wout_ref, bout_ref,
                  v_ref, wt_ref, out_ref):
        def elu(x):
            return jnp.where(x > 0.0, x, jnp.exp(jnp.minimum(x, 0.0)) - 1.0)

        def mm(a, b):
            return jnp.dot(a, b, preferred_element_type=jnp.float32)

        wsem = wsem_ref[...]
        bsem = bsem_ref[...]
        qsem = qsem_ref[...]
        v0 = v_ref[0, 0]
        vrest = jnp.sum(v_ref[...]) - v0
        wt = wt_ref[0, 0]

        rs = []
        for i in range(2):
            out0_l, outm_l, wm_l = [], [], []
            for t in range(_NT):
                it = i * _NT + t
                o0 = elu(zc_ref[:, it * _D:(it + 1) * _D])
                om = elu(zc_ref[:, 96 + it * _D:96 + (it + 1) * _D])
                y0 = jnp.tanh(mm(o0, wsem) + bsem)
                ym = jnp.tanh(mm(om, wsem) + bsem)
                w0 = mm(y0, qsem)
                wm = mm(ym, qsem)
                wm_l.append((w0 + 7.0 * wm) * 0.125)
                out0_l.append(o0)
                outm_l.append(om)
            wmean = jnp.concatenate(wm_l, axis=1)
            mmax = jnp.max(wmean, axis=1, keepdims=True)
            be = jnp.exp(wmean - mmax)
            beta = be / jnp.sum(be, axis=1, keepdims=True)
            hp = beta[:, 0:1] * (v0 * out0_l[0] + vrest * outm_l[0])
            for t in range(1, _NT):
                hp = hp + beta[:, t:t + 1] * (v0 * out0_l[t] + vrest * outm_l[t])
            hp = hp * wt
            ne = zc_ref[:, 192 + i * _D:192 + (i + 1) * _D]
            ctx = jnp.concatenate([ne, hp], axis=1)
            r = mm(ctx, wout_ref[...]) + bout_ref[...]
            rs.append(r)

        num = jnp.sum(rs[0] * rs[1], axis=1, keepdims=True)
        n0 = jnp.sqrt(jnp.sum(rs[0] * rs[0], axis=1, keepdims=True))
        n1 = jnp.sqrt(jnp.sum(rs[1] * rs[1], axis=1, keepdims=True))
        out_ref[...] = num / jnp.maximum(n0 * n1, 1e-8)

    return pl.pallas_call(
        post_body,
        out_shape=jax.ShapeDtypeStruct((B, 1), jnp.float32),
    )


@jax.jit
def _run(nodes, features, emb_table, W_gat, a_gat, W_sem, b_sem, q_sem,
         W_out, b_out, v, weight):
    B = nodes.shape[0]
    wpk = jnp.concatenate([emb_table, W_gat, a_gat.reshape(2, _D)], axis=0)
    zc = _make_sc_main(B)(nodes, features.reshape(B, _FC), wpk)
    sim = _make_post(B)(
        zc, W_sem, b_sem.reshape(1, _D), q_sem.reshape(_D, 1), W_out,
        b_out.reshape(1, 300), v.reshape(1, _NR), weight.reshape(1, 1))
    return sim.reshape(B)


def kernel(nodes, features, prop_nodes, prop_features, max_prop_len, emb_table,
           W_gat, a_gat, W_sem, b_sem, q_sem, W_out, b_out, v, weight):
    return _run(nodes, features, emb_table, W_gat, a_gat, W_sem, b_sem,
                q_sem, W_out, b_out, v, weight)


# speedup vs baseline: 1.1626x; 1.1626x over previous
import functools

import jax
import jax.numpy as jnp
from jax import lax
from jax.experimental import pallas as pl
from jax.experimental.pallas import tpu as pltpu
from jax.experimental.pallas import tpu_sc as plsc

_VOCAB = 32
_D = 16
_NT = 3
_NR = 8
_NW = 32
_LANES = 16
_FC = 2 * _NT * 4 * 7
_ZR = 14 * _D


def _make_sc_main(B):
    chunk = B // _NW
    ngrp = chunk // _LANES
    mesh = plsc.VectorSubcoreMesh(core_axis_name="c", subcore_axis_name="s")

    @functools.partial(
        pl.kernel,
        mesh=mesh,
        compiler_params=pltpu.CompilerParams(use_tc_tiling_on_sc=False,
                                             needs_layout_passes=False),
        out_type=jax.ShapeDtypeStruct((_NW, _ZR, 32), jnp.float32),
        scratch_types=[
            pltpu.VMEM((chunk, 2), jnp.int32),
            pltpu.VMEM((chunk, _FC), jnp.int32),
            pltpu.VMEM((50, _D), jnp.float32),
            pltpu.VMEM((_VOCAB * _D,), jnp.float32),
            pltpu.VMEM((2 * _VOCAB,), jnp.float32),
            pltpu.VMEM((_ZR, chunk), jnp.float32),
        ],
    )
    def sc_main(nodes_hbm, feats_hbm, wpk_hbm, zc_hbm,
                nodes_v, feats_v, wpk_v, we_v, f12_v, zc_v):
        wid = lax.axis_index("s") * 2 + lax.axis_index("c")
        base = wid * chunk
        with jax.named_scope("stage_in"):
            pltpu.sync_copy(nodes_hbm.at[pl.ds(base, chunk)], nodes_v)
            pltpu.sync_copy(feats_hbm.at[pl.ds(base, chunk)], feats_v)
            pltpu.sync_copy(wpk_hbm, wpk_v)

        lane = lax.iota(jnp.int32, _LANES)
        zero16 = jnp.zeros((_LANES,), jnp.float32)

        def spl(x):
            return jnp.full((_LANES,), x, jnp.int32)

        tb_scope = jax.named_scope("table_build")
        tb_scope.__enter__()
        vvec = [lane, lane + _LANES]
        f1h = [zero16, zero16]
        f2h = [zero16, zero16]
        wgat_rows = [wpk_v[_VOCAB + k] for k in range(_D)]
        a1row = wpk_v[48]
        a2row = wpk_v[49]
        embcol = [[plsc.load_gather(wpk_v, [vvec[h], spl(k)]) for h in range(2)]
                  for k in range(_D)]
        for d in range(_D):
            a1d = a1row[d]
            a2d = a2row[d]
            for h in range(2):
                acc = embcol[0][h] * wgat_rows[0][d]
                for k in range(1, _D):
                    acc = acc + embcol[k][h] * wgat_rows[k][d]
                plsc.store_scatter(we_v, [vvec[h] * _D + d], acc)
                f1h[h] = f1h[h] + acc * a1d
                f2h[h] = f2h[h] + acc * a2d
        half = [lane, lane + _LANES]
        for h in range(2):
            plsc.store_scatter(f12_v, [half[h]], f1h[h])
            plsc.store_scatter(f12_v, [half[h] + _VOCAB], f2h[h])
        tb_scope.__exit__(None, None, None)

        def wtree8(w, xs):
            p = [w[r] * xs[r] for r in range(8)]
            return ((p[0] + p[1]) + (p[2] + p[3])) + \
                   ((p[4] + p[5]) + (p[6] + p[7]))

        def tree8(xs):
            return ((xs[0] + xs[1]) + (xs[2] + xs[3])) + \
                   ((xs[4] + xs[5]) + (xs[6] + xs[7]))

        with jax.named_scope("node_emb"):
         for g in range(ngrp):
            col = g * _LANES + lane
            for i in range(2):
                nid = plsc.load_gather(nodes_v, [col, spl(i)])
                for d in range(_D):
                    ne = plsc.load_gather(wpk_v, [nid, spl(d)])
                    plsc.store_scatter(zc_v, [spl(192 + i * _D + d), col], ne)

        @plsc.parallel_loop(0, ngrp * 2 * _NT, unroll=2)
        def body(k, carry=None):
            g = k // (2 * _NT)
            it = k % (2 * _NT)
            i = it // _NT
            col = g * _LANES + lane
            fb = it * 28

            ids8 = [plsc.load_gather(nodes_v, [col, spl(i)])]
            for r in range(1, _NR):
                ids8.append(plsc.load_gather(feats_v, [col, spl(fb + r - 1)]))

            f1_0 = plsc.load_gather(f12_v, [ids8[0]])
            f2 = [plsc.load_gather(f12_v, [ids8[r] + _VOCAB])
                  for r in range(_NR)]
            e = [jnp.where(x >= 0.0, x, 0.2 * x) for x in
                 [f1_0 + f2r for f2r in f2]]
            m = jnp.maximum(jnp.maximum(jnp.maximum(e[0], e[1]),
                                        jnp.maximum(e[2], e[3])),
                            jnp.maximum(jnp.maximum(e[4], e[5]),
                                        jnp.maximum(e[6], e[7])))
            ex = [jnp.exp(er - m) for er in e]
            inv = 1.0 / tree8(ex)
            attn = [exr * inv for exr in ex]

            cids = [ids8[1]]
            for p in range(1, 4):
                cids.append(plsc.load_gather(feats_v, [col, spl(fb + p * 7)]))
            cb = [plsc.load_gather(wpk_v, [c, spl(0)]) != 0.0 for c in cids]
            cacc = jnp.logical_or(jnp.logical_or(cb[0], cb[1]),
                                  jnp.logical_or(cb[2], cb[3]))

            base_r = [ids8[r] * _D for r in range(_NR)]
            row0 = spl(it * _D)
            for d in range(_D):
                rows = [plsc.load_gather(we_v, [base_r[r] + d])
                        for r in range(_NR)]
                z0d = jnp.where(cacc, wtree8(attn, rows), zero16)
                zmd = jnp.where(cacc, tree8(rows) * 0.125, zero16)
                plsc.store_scatter(zc_v, [row0 + d, col], z0d)
                plsc.store_scatter(zc_v, [row0 + (96 + d), col], zmd)

        with jax.named_scope("stage_out"):
            pltpu.sync_copy(zc_v, zc_hbm.at[wid])

    return sc_main


def _make_post(B):
    def post_body(zc_ref, wsem_ref, bsem_ref, qsem_ref, wout_ref, bout_ref,
                  v_ref, wt_ref, out_ref):
        def elu(x):
            return jnp.where(x > 0.0, x, jnp.exp(jnp.minimum(x, 0.0)) - 1.0)

        def dot0(a, b):
            return lax.dot_general(a, b, (((0,), (0,)), ((), ())),
                                   preferred_element_type=jnp.float32)

        wsem = wsem_ref[...]
        bsem = bsem_ref[...]
        qsemT = qsem_ref[...]
        v0 = v_ref[0, 0]
        vrest = jnp.sum(v_ref[...]) - v0
        wt = wt_ref[0, 0]

        rs = []
        for i in range(2):
            out0_l, outm_l, wm_l = [], [], []
            for t in range(_NT):
                it = i * _NT + t
                o0 = elu(zc_ref[it * _D:(it + 1) * _D, :])
                om = elu(zc_ref[96 + it * _D:96 + (it + 1) * _D, :])
                y0 = jnp.tanh(dot0(wsem, o0) + bsem)
                ym = jnp.tanh(dot0(wsem, om) + bsem)
                w0 = dot0(qsemT, y0)
                wm = dot0(qsemT, ym)
                wm_l.append((w0 + 7.0 * wm) * 0.125)
                out0_l.append(o0)
                outm_l.append(om)
            wmean = jnp.concatenate(wm_l, axis=0)
            mm = jnp.max(wmean, axis=0, keepdims=True)
            be = jnp.exp(wmean - mm)
            beta = be / jnp.sum(be, axis=0, keepdims=True)
            hp = beta[0:1] * (v0 * out0_l[0] + vrest * outm_l[0])
            for t in range(1, _NT):
                hp = hp + beta[t:t + 1] * (v0 * out0_l[t] + vrest * outm_l[t])
            hp = hp * wt
            ne = zc_ref[192 + i * _D:192 + (i + 1) * _D, :]
            ctx = jnp.concatenate([ne, hp], axis=0)
            r = dot0(wout_ref[...], ctx) + bout_ref[...]
            rs.append(r)

        num = jnp.sum(rs[0] * rs[1], axis=0, keepdims=True)
        n0 = jnp.sqrt(jnp.sum(rs[0] * rs[0], axis=0, keepdims=True))
        n1 = jnp.sqrt(jnp.sum(rs[1] * rs[1], axis=0, keepdims=True))
        out_ref[...] = num / jnp.maximum(n0 * n1, 1e-8)

    return pl.pallas_call(
        post_body,
        out_shape=jax.ShapeDtypeStruct((1, B), jnp.float32),
    )


@jax.jit
def _run(nodes, features, emb_table, W_gat, a_gat, W_sem, b_sem, q_sem,
         W_out, b_out, v, weight):
    B = nodes.shape[0]
    wpk = jnp.concatenate([emb_table, W_gat, a_gat.reshape(2, _D)], axis=0)
    zc = _make_sc_main(B)(nodes, features.reshape(B, _FC), wpk).transpose(1, 0, 2).reshape(_ZR, B)
    sim = _make_post(B)(
        zc, W_sem, b_sem.reshape(_D, 1), q_sem.reshape(_D, 1), W_out,
        b_out.reshape(300, 1), v.reshape(1, _NR), weight.reshape(1, 1))
    return sim.reshape(B)


def kernel(nodes, features, prop_nodes, prop_features, max_prop_len, emb_table,
           W_gat, a_gat, W_sem, b_sem, q_sem, W_out, b_out, v, weight):
    return _run(nodes, features, emb_table, W_gat, a_gat, W_sem, b_sem,
                q_sem, W_out, b_out, v, weight)


# speedup vs baseline: 1.2629x; 1.0863x over previous
import functools

import jax
import jax.numpy as jnp
from jax import lax
from jax.experimental import pallas as pl
from jax.experimental.pallas import tpu as pltpu
from jax.experimental.pallas import tpu_sc as plsc

_VOCAB = 32
_D = 16
_NT = 3
_NR = 8
_NW = 32
_LANES = 16
_FC = 2 * _NT * 4 * 7
_ZR = 14 * _D


def _make_sc_main(B):
    chunk = B // _NW
    ngrp = chunk // _LANES
    mesh = plsc.VectorSubcoreMesh(core_axis_name="c", subcore_axis_name="s")

    @functools.partial(
        pl.kernel,
        mesh=mesh,
        compiler_params=pltpu.CompilerParams(use_tc_tiling_on_sc=False,
                                             needs_layout_passes=False),
        out_type=jax.ShapeDtypeStruct((_ZR, B), jnp.float32),
        scratch_types=[
            pltpu.VMEM((chunk, 2), jnp.int32),
            pltpu.VMEM((chunk, _FC), jnp.int32),
            pltpu.VMEM((50, _D), jnp.float32),
            pltpu.VMEM((_VOCAB * _D,), jnp.float32),
            pltpu.VMEM((2 * _VOCAB,), jnp.float32),
            pltpu.VMEM((_ZR, chunk), jnp.float32),
            pltpu.SemaphoreType.DMA,
            pltpu.SemaphoreType.DMA,
            pltpu.SemaphoreType.DMA,
        ],
    )
    def sc_main(nodes_hbm, feats_hbm, wpk_hbm, zc_hbm,
                nodes_v, feats_v, wpk_v, we_v, f12_v, zc_v,
                sem0, sem1, sem2):
        wid = lax.axis_index("s") * 2 + lax.axis_index("c")
        base = wid * chunk
        with jax.named_scope("stage_in"):
            c0 = pltpu.async_copy(wpk_hbm, wpk_v, sem0)
            c1 = pltpu.async_copy(nodes_hbm.at[pl.ds(base, chunk)], nodes_v,
                                  sem1)
            c2 = pltpu.async_copy(feats_hbm.at[pl.ds(base, chunk)], feats_v,
                                  sem2)
            c0.wait()
            c1.wait()
            c2.wait()

        lane = lax.iota(jnp.int32, _LANES)
        zero16 = jnp.zeros((_LANES,), jnp.float32)

        def spl(x):
            return jnp.full((_LANES,), x, jnp.int32)

        tb_scope = jax.named_scope("table_build")
        tb_scope.__enter__()
        vvec = [lane, lane + _LANES]
        f1h = [zero16, zero16]
        f2h = [zero16, zero16]
        wgat_rows = [wpk_v[_VOCAB + k] for k in range(_D)]
        a1row = wpk_v[48]
        a2row = wpk_v[49]
        embcol = [[plsc.load_gather(wpk_v, [vvec[h], spl(k)]) for h in range(2)]
                  for k in range(_D)]
        for d in range(_D):
            a1d = a1row[d]
            a2d = a2row[d]
            for h in range(2):
                acc = embcol[0][h] * wgat_rows[0][d]
                for k in range(1, _D):
                    acc = acc + embcol[k][h] * wgat_rows[k][d]
                plsc.store_scatter(we_v, [vvec[h] * _D + d], acc)
                f1h[h] = f1h[h] + acc * a1d
                f2h[h] = f2h[h] + acc * a2d
        half = [lane, lane + _LANES]
        for h in range(2):
            plsc.store_scatter(f12_v, [half[h]], f1h[h])
            plsc.store_scatter(f12_v, [half[h] + _VOCAB], f2h[h])
        tb_scope.__exit__(None, None, None)

        def wtree8(w, xs):
            p = [w[r] * xs[r] for r in range(8)]
            return ((p[0] + p[1]) + (p[2] + p[3])) + \
                   ((p[4] + p[5]) + (p[6] + p[7]))

        def tree8(xs):
            return ((xs[0] + xs[1]) + (xs[2] + xs[3])) + \
                   ((xs[4] + xs[5]) + (xs[6] + xs[7]))

        with jax.named_scope("node_emb"):
         for g in range(ngrp):
            col = g * _LANES + lane
            for i in range(2):
                nid = plsc.load_gather(nodes_v, [col, spl(i)])
                for d in range(_D):
                    ne = plsc.load_gather(wpk_v, [nid, spl(d)])
                    plsc.store_scatter(zc_v, [spl(192 + i * _D + d), col], ne)

        @plsc.parallel_loop(0, ngrp * 2 * _NT, unroll=2)
        def body(k, carry=None):
            g = k // (2 * _NT)
            it = k % (2 * _NT)
            i = it // _NT
            col = g * _LANES + lane
            fb = it * 28

            ids8 = [plsc.load_gather(nodes_v, [col, spl(i)])]
            for r in range(1, _NR):
                ids8.append(plsc.load_gather(feats_v, [col, spl(fb + r - 1)]))

            f1_0 = plsc.load_gather(f12_v, [ids8[0]])
            f2 = [plsc.load_gather(f12_v, [ids8[r] + _VOCAB])
                  for r in range(_NR)]
            e = [jnp.where(x >= 0.0, x, 0.2 * x) for x in
                 [f1_0 + f2r for f2r in f2]]
            m = jnp.maximum(jnp.maximum(jnp.maximum(e[0], e[1]),
                                        jnp.maximum(e[2], e[3])),
                            jnp.maximum(jnp.maximum(e[4], e[5]),
                                        jnp.maximum(e[6], e[7])))
            ex = [jnp.exp(er - m) for er in e]
            inv = 1.0 / tree8(ex)
            attn = [exr * inv for exr in ex]

            cids = [ids8[1]]
            for p in range(1, 4):
                cids.append(plsc.load_gather(feats_v, [col, spl(fb + p * 7)]))
            cb = [plsc.load_gather(wpk_v, [c, spl(0)]) != 0.0 for c in cids]
            cacc = jnp.logical_or(jnp.logical_or(cb[0], cb[1]),
                                  jnp.logical_or(cb[2], cb[3]))

            base_r = [ids8[r] * _D for r in range(_NR)]
            row0 = spl(it * _D)
            for d in range(_D):
                rows = [plsc.load_gather(we_v, [base_r[r] + d])
                        for r in range(_NR)]
                z0d = jnp.where(cacc, wtree8(attn, rows), zero16)
                zmd = jnp.where(cacc, tree8(rows) * 0.125, zero16)
                plsc.store_scatter(zc_v, [row0 + d, col], z0d)
                plsc.store_scatter(zc_v, [row0 + (96 + d), col], zmd)

        with jax.named_scope("stage_out"):
            pltpu.sync_copy(zc_v, zc_hbm.at[:, pl.ds(base, chunk)])

    return sc_main


def _make_post(B):
    def post_body(zc_ref, wsem_ref, bsem_ref, qsem_ref, wout_ref, bout_ref,
                  v_ref, wt_ref, out_ref):
        def elu(x):
            return jnp.where(x > 0.0, x, jnp.exp(jnp.minimum(x, 0.0)) - 1.0)

        def dot0(a, b):
            return lax.dot_general(a, b, (((0,), (0,)), ((), ())),
                                   preferred_element_type=jnp.float32)

        wsem = wsem_ref[...]
        bsem = bsem_ref[...]
        qsemT = qsem_ref[...]
        v0 = v_ref[0, 0]
        vrest = jnp.sum(v_ref[...]) - v0
        wt = wt_ref[0, 0]

        rs = []
        for i in range(2):
            out0_l, outm_l, wm_l = [], [], []
            for t in range(_NT):
                it = i * _NT + t
                o0 = elu(zc_ref[it * _D:(it + 1) * _D, :])
                om = elu(zc_ref[96 + it * _D:96 + (it + 1) * _D, :])
                y0 = jnp.tanh(dot0(wsem, o0) + bsem)
                ym = jnp.tanh(dot0(wsem, om) + bsem)
                w0 = dot0(qsemT, y0)
                wm = dot0(qsemT, ym)
                wm_l.append((w0 + 7.0 * wm) * 0.125)
                out0_l.append(o0)
                outm_l.append(om)
            wmean = jnp.concatenate(wm_l, axis=0)
            mm = jnp.max(wmean, axis=0, keepdims=True)
            be = jnp.exp(wmean - mm)
            beta = be / jnp.sum(be, axis=0, keepdims=True)
            hp = beta[0:1] * (v0 * out0_l[0] + vrest * outm_l[0])
            for t in range(1, _NT):
                hp = hp + beta[t:t + 1] * (v0 * out0_l[t] + vrest * outm_l[t])
            hp = hp * wt
            ne = zc_ref[192 + i * _D:192 + (i + 1) * _D, :]
            ctx = jnp.concatenate([ne, hp], axis=0)
            r = dot0(wout_ref[...], ctx) + bout_ref[...]
            rs.append(r)

        num = jnp.sum(rs[0] * rs[1], axis=0, keepdims=True)
        n0 = jnp.sqrt(jnp.sum(rs[0] * rs[0], axis=0, keepdims=True))
        n1 = jnp.sqrt(jnp.sum(rs[1] * rs[1], axis=0, keepdims=True))
        out_ref[...] = num / jnp.maximum(n0 * n1, 1e-8)

    return pl.pallas_call(
        post_body,
        out_shape=jax.ShapeDtypeStruct((1, B), jnp.float32),
    )


@jax.jit
def _run(nodes, features, emb_table, W_gat, a_gat, W_sem, b_sem, q_sem,
         W_out, b_out, v, weight):
    B = nodes.shape[0]
    wpk = jnp.concatenate([emb_table, W_gat, a_gat.reshape(2, _D)], axis=0)
    zc = _make_sc_main(B)(nodes, features.reshape(B, _FC), wpk)
    sim = _make_post(B)(
        zc, W_sem, b_sem.reshape(_D, 1), q_sem.reshape(_D, 1), W_out,
        b_out.reshape(300, 1), v.reshape(1, _NR), weight.reshape(1, 1))
    return sim.reshape(B)


def kernel(nodes, features, prop_nodes, prop_features, max_prop_len, emb_table,
           W_gat, a_gat, W_sem, b_sem, q_sem, W_out, b_out, v, weight):
    return _run(nodes, features, emb_table, W_gat, a_gat, W_sem, b_sem,
                q_sem, W_out, b_out, v, weight)
